# baseline (speedup 1.0000x reference)
import functools

import jax
import jax.numpy as jnp
from jax import lax
from jax.experimental import pallas as pl
from jax.experimental.pallas import tpu as pltpu
from jax.experimental.pallas import tpu_sc as plsc

B = 16384
F = 26
D = 16
BF = B * F
NC, NS, L = 2, 16, 16
NW = NC * NS
PER_W = BF // NW
CHUNK = 128
NCH = PER_W // CHUNK
OFF_PERIOD = 208
OFF_LEN = OFF_PERIOD + CHUNK

_mesh = plsc.VectorSubcoreMesh(core_axis_name="c", subcore_axis_name="s")


@functools.partial(
    pl.kernel,
    out_type=jax.ShapeDtypeStruct((BF, D), jnp.float32),
    mesh=_mesh,
    scratch_types=[
        pltpu.VMEM((NCH, CHUNK), jnp.int32),
        pltpu.VMEM((OFF_LEN,), jnp.int32),
        pltpu.VMEM((2, CHUNK, D), jnp.float32),
        pltpu.SemaphoreType.DMA,
        pltpu.SemaphoreType.DMA,
    ],
    compiler_params=pltpu.CompilerParams(use_tc_tiling_on_sc=False),
)
def _embed(x_hbm, table_hbm, offs_hbm, out_hbm, idx_v, offs_v, rows_v, gsem, osem):
    wid = lax.axis_index("s") * NC + lax.axis_index("c")
    base = wid * PER_W

    pltpu.sync_copy(x_hbm.at[pl.ds(wid * NCH, NCH)], idx_v)
    pltpu.sync_copy(offs_hbm, offs_v)

    def add_body(c, carry):
        r0 = lax.rem(c * CHUNK, OFF_PERIOD)
        for k in range(CHUNK // L):
            cur = idx_v[c, pl.ds(k * L, L)]
            off = offs_v[pl.ds(r0 + k * L, L)]
            idx_v[c, pl.ds(k * L, L)] = cur + off
        return carry

    lax.fori_loop(0, NCH, add_body, 0)

    def g_body(c, carry):
        rows = rows_v.at[0]
        pltpu.async_copy(table_hbm.at[idx_v.at[c]], rows, gsem).wait()
        pltpu.sync_copy(rows, out_hbm.at[pl.ds(base + c * CHUNK, CHUNK)])
        return carry

    lax.fori_loop(0, NCH, g_body, 0)


def kernel(x_cat, table, offsets):
    offs_tiled = offsets[jnp.arange(OFF_LEN, dtype=jnp.int32) % F]
    x_flat = x_cat.reshape(NW * NCH, CHUNK)
    out = _embed(x_flat, table, offs_tiled)
    return out.reshape(B, F, D)


# speedup vs baseline: 1.0763x; 1.0763x over previous
import functools

import jax
import jax.numpy as jnp
from jax import lax
from jax.experimental import pallas as pl
from jax.experimental.pallas import tpu as pltpu
from jax.experimental.pallas import tpu_sc as plsc

B = 16384
F = 26
D = 16
BF = B * F
NC, NS, L = 2, 16, 16
NW = NC * NS
PER_W = BF // NW
CHUNK = 128
NCH = PER_W // CHUNK
OFF_PERIOD = 208
OFF_LEN = OFF_PERIOD + CHUNK
G = 8
NG = NCH // G
GROUP_ROWS = G * CHUNK

_mesh = plsc.VectorSubcoreMesh(core_axis_name="c", subcore_axis_name="s")


@functools.partial(
    pl.kernel,
    out_type=jax.ShapeDtypeStruct((BF, D), jnp.float32),
    mesh=_mesh,
    scratch_types=[
        pltpu.VMEM((NCH, CHUNK), jnp.int32),
        pltpu.VMEM((OFF_LEN,), jnp.int32),
        pltpu.VMEM((2, GROUP_ROWS, D), jnp.float32),
        pltpu.SemaphoreType.DMA,
    ],
    compiler_params=pltpu.CompilerParams(use_tc_tiling_on_sc=False),
)
def _embed(x_hbm, table_hbm, offs_hbm, out_hbm, idx_v, offs_v, rows_v, gsem):
    wid = lax.axis_index("s") * NC + lax.axis_index("c")
    base = wid * PER_W

    pltpu.sync_copy(x_hbm.at[pl.ds(wid * NCH, NCH)], idx_v)
    pltpu.sync_copy(offs_hbm, offs_v)

    def add_group(g):
        def body(c, carry):
            r0 = lax.rem(c * CHUNK, OFF_PERIOD)
            for k in range(CHUNK // L):
                cur = idx_v[c, pl.ds(k * L, L)]
                off = offs_v[pl.ds(r0 + k * L, L)]
                idx_v[c, pl.ds(k * L, L)] = cur + off
            return carry

        lax.fori_loop(g * G, (g + 1) * G, body, 0)

    def fire_group(g, p):
        def body(i, carry):
            c = g * G + i
            pltpu.async_copy(
                table_hbm.at[idx_v.at[c]],
                rows_v.at[p, pl.ds(i * CHUNK, CHUNK)],
                gsem,
            )
            return carry

        lax.fori_loop(0, G, body, 0)

    def out_slice(g):
        return out_hbm.at[pl.ds(base + g * GROUP_ROWS, GROUP_ROWS)]

    def drain_group(g, p):
        pltpu.make_async_copy(out_slice(g), rows_v.at[p], gsem).wait()

    add_group(0)
    fire_group(0, 0)

    def loop_body(g, carry):
        p = lax.rem(g, 2)

        @pl.when(g + 1 < NG)
        def _():
            add_group(g + 1)
            fire_group(g + 1, 1 - p)

        drain_group(g, p)
        pltpu.sync_copy(rows_v.at[p], out_slice(g))
        return carry

    lax.fori_loop(0, NG, loop_body, 0)


def kernel(x_cat, table, offsets):
    offs_tiled = offsets[jnp.arange(OFF_LEN, dtype=jnp.int32) % F]
    x_flat = x_cat.reshape(NW * NCH, CHUNK)
    out = _embed(x_flat, table, offs_tiled)
    return out.reshape(B, F, D)


# speedup vs baseline: 1.2073x; 1.1217x over previous
import functools

import jax
import jax.numpy as jnp
from jax import lax
from jax.experimental import pallas as pl
from jax.experimental.pallas import tpu as pltpu
from jax.experimental.pallas import tpu_sc as plsc

B = 16384
F = 26
D = 16
BF = B * F
NC, NS, L = 2, 16, 16
NW = NC * NS
PER_W = BF // NW
CHUNK = 128
NCH = PER_W // CHUNK
OFF_PERIOD = 208
OFF_LEN = 384
G = 8
NG = NCH // G
GW = G * CHUNK * D
TOTALG = 1040000 * D // 128

_mesh = plsc.VectorSubcoreMesh(core_axis_name="c", subcore_axis_name="s")


@functools.partial(
    pl.kernel,
    out_type=jax.ShapeDtypeStruct((BF * D,), jnp.float32),
    mesh=_mesh,
    scratch_types=[
        pltpu.VMEM((PER_W,), jnp.int32),
        pltpu.VMEM((PER_W,), jnp.int32),
        pltpu.VMEM((OFF_LEN,), jnp.int32),
        pltpu.VMEM((2 * CHUNK, 128), jnp.float32),
        pltpu.VMEM((2 * GW,), jnp.float32),
        pltpu.SemaphoreType.DMA,
        pltpu.SemaphoreType.DMA,
    ],
    compiler_params=pltpu.CompilerParams(needs_layout_passes=False),
)
def _embed(x_hbm, table_hbm, offs_hbm, out_hbm, xidx_v, gidx_v, offs_v,
           big_v, stage_v, gsem, osem):
    wid = lax.axis_index("s") * NC + lax.axis_index("c")
    base = wid * PER_W

    pltpu.sync_copy(x_hbm.at[pl.ds(base, PER_W)], xidx_v)
    pltpu.sync_copy(offs_hbm, offs_v)

    def add_body(c, carry):
        r0 = lax.rem(c * CHUNK, OFF_PERIOD)
        j0 = c * CHUNK
        for k in range(CHUNK // L):
            cur = xidx_v[pl.ds(j0 + k * L, L)]
            off = offs_v[pl.ds(r0 + k * L, L)]
            t = cur + off
            xidx_v[pl.ds(j0 + k * L, L)] = t
            gidx_v[pl.ds(j0 + k * L, L)] = lax.shift_right_logical(t, 3)
        return carry

    lax.fori_loop(0, NCH, add_body, 0)

    iota = lax.iota(jnp.int32, L)
    iota16 = iota * D

    def bigbuf(c):
        return big_v.at[pl.ds(lax.rem(c, 2) * CHUNK, CHUNK)]

    def fire(c):
        pltpu.async_copy(
            table_hbm.at[gidx_v.at[pl.ds(c * CHUNK, CHUNK)]], bigbuf(c), gsem
        )

    def drain_gather(c):
        pltpu.make_async_copy(
            table_hbm.at[pl.ds(0, CHUNK)], bigbuf(c), gsem
        ).wait()

    def extract(c, sp):
        src = bigbuf(c)
        stagebuf = stage_v.at[pl.ds(sp * GW, GW)]
        q = lax.rem(c, G) * (CHUNK * D)

        def block(b, carry):
            j0 = b * L
            v = xidx_v[pl.ds(c * CHUNK + j0, L)]
            sub16 = (v & 7) * D
            row = iota + j0
            posb = q + j0 * D
            for d in range(D):
                col = sub16 + d
                vals = plsc.load_gather(src, [row, col])
                sidx = iota16 + (posb + d)
                plsc.store_scatter(stagebuf, [sidx], vals)
            return carry

        lax.fori_loop(0, CHUNK // L, block, 0)

    def out_slice(g):
        return out_hbm.at[pl.ds(base * D + g * GW, GW)]

    def stage_slice(g):
        return stage_v.at[pl.ds(lax.rem(g, 2) * GW, GW)]

    fire(0)

    def group_body(g, carry):
        sp = lax.rem(g, 2)

        @pl.when(g >= 2)
        def _():
            pltpu.make_async_copy(stage_slice(g), out_slice(g), osem).wait()

        def chunk_body(cc, carry2):
            c = g * G + cc

            @pl.when(c + 1 < NCH)
            def _():
                fire(c + 1)

            drain_gather(c)
            extract(c, sp)
            return carry2

        lax.fori_loop(0, G, chunk_body, 0)
        pltpu.async_copy(stage_slice(g), out_slice(g), osem)
        return carry

    lax.fori_loop(0, NG, group_body, 0)

    for g in (NG - 2, NG - 1):
        pltpu.make_async_copy(stage_slice(g), out_slice(g), osem).wait()


def kernel(x_cat, table, offsets):
    offs_tiled = offsets[jnp.arange(OFF_LEN, dtype=jnp.int32) % F]
    x_flat = x_cat.reshape(BF)
    table128 = table.reshape(TOTALG, 128)
    out = _embed(x_flat, table128, offs_tiled)
    return out.reshape(B, F, D)


# speedup vs baseline: 1.7974x; 1.4887x over previous
import functools

import jax
import jax.numpy as jnp
from jax import lax
from jax.experimental import pallas as pl
from jax.experimental.pallas import tpu as pltpu
from jax.experimental.pallas import tpu_sc as plsc

B = 16384
F = 26
D = 16
BF = B * F
NC, NS, L = 2, 16, 16
NW = NC * NS
PER_W = BF // NW
CHUNK = 128
NCH = PER_W // CHUNK
CW = CHUNK * D
HW = CW // 2
FS = B * D
DHS = B * D // 2

_mesh = plsc.VectorSubcoreMesh(core_axis_name="c", subcore_axis_name="s")


@functools.partial(
    pl.kernel,
    out_type=jax.ShapeDtypeStruct((BF * D,), jnp.float32),
    mesh=_mesh,
    scratch_types=[
        pltpu.VMEM((PER_W,), jnp.int32),
        pltpu.VMEM((32,), jnp.int32),
        pltpu.VMEM((2 * CHUNK, D), jnp.float32),
        pltpu.VMEM((2 * CW,), jnp.float32),
        pltpu.SemaphoreType.DMA,
        pltpu.SemaphoreType.DMA,
    ],
    compiler_params=pltpu.CompilerParams(
        use_tc_tiling_on_sc=False, needs_layout_passes=False
    ),
)
def _embed(x_hbm, table_hbm, offs_hbm, out_hbm, xv, offs_v, big_v, stage_v,
           gsem, osem):
    wid = lax.axis_index("s") * NC + lax.axis_index("c")
    base = wid * PER_W

    pltpu.sync_copy(x_hbm.at[pl.ds(base, PER_W)], xv)
    pltpu.sync_copy(offs_hbm, offs_v)

    def add_body(i, carry):
        f = lax.shift_right_logical(base + i * L, 14)
        off = plsc.load_gather(offs_v, [lax.broadcast(f, (L,))])
        s = pl.ds(i * L, L)
        xv[s] = xv[s] + off
        return carry

    lax.fori_loop(0, PER_W // L, add_body, 0)

    iota = lax.iota(jnp.int32, L)

    def bigbuf(c):
        return big_v.at[pl.ds(lax.rem(c, 2) * CHUNK, CHUNK)]

    def fire(c):
        pltpu.async_copy(
            table_hbm.at[xv.at[pl.ds(c * CHUNK, CHUNK)]], bigbuf(c), gsem
        )

    def drain_gather(c):
        pltpu.make_async_copy(
            table_hbm.at[pl.ds(0, CHUNK)], bigbuf(c), gsem
        ).wait()

    def extract(c):
        src = bigbuf(c)
        sbuf = lax.rem(c, 2) * CW

        def block(b, carry):
            row = iota + b * L
            for d in range(D):
                vals = plsc.load_gather(src, [row, lax.broadcast(d, (L,))])
                stage_v[pl.ds(sbuf + d * CHUNK + b * L, L)] = vals
            return carry

        lax.fori_loop(0, CHUNK // L, block, 0)

    def out_half(c, dh):
        j0 = base + c * CHUNK
        f = lax.shift_right_logical(j0, 14)
        bt = lax.shift_right_logical(lax.rem(j0, B), 7)
        return out_hbm.at[pl.ds(f * FS + dh * DHS + bt * HW, HW)]

    def stage_half(c, dh):
        return stage_v.at[pl.ds(lax.rem(c, 2) * CW + dh * HW, HW)]

    fire(0)

    def chunk_body(c, carry):
        @pl.when(c + 1 < NCH)
        def _():
            fire(c + 1)

        @pl.when(c >= 2)
        def _():
            pltpu.make_async_copy(stage_half(c, 0), out_half(c, 0), osem).wait()
            pltpu.make_async_copy(stage_half(c, 1), out_half(c, 1), osem).wait()

        drain_gather(c)
        extract(c)
        pltpu.async_copy(stage_half(c, 0), out_half(c, 0), osem)
        pltpu.async_copy(stage_half(c, 1), out_half(c, 1), osem)
        return carry

    lax.fori_loop(0, NCH, chunk_body, 0)

    for c in (NCH - 2, NCH - 1):
        pltpu.make_async_copy(stage_half(c, 0), out_half(c, 0), osem).wait()
        pltpu.make_async_copy(stage_half(c, 1), out_half(c, 1), osem).wait()


def kernel(x_cat, table, offsets):
    xT = jnp.transpose(x_cat).reshape(BF)
    offs_pad = jnp.concatenate([offsets, jnp.zeros((32 - F,), jnp.int32)])
    out = _embed(xT, table, offs_pad)
    out5 = out.reshape(F, 2, B // 128, D // 2, 128)
    return out5.transpose(2, 4, 0, 1, 3).reshape(B, F, D)


# speedup vs baseline: 1.8059x; 1.0048x over previous
import functools

import jax
import jax.numpy as jnp
from jax import lax
from jax.experimental import pallas as pl
from jax.experimental.pallas import tpu as pltpu
from jax.experimental.pallas import tpu_sc as plsc

B = 16384
F = 26
D = 16
BF = B * F
NC, NS, L = 2, 16, 16
NW = NC * NS
PER_W = BF // NW
CHUNK = 128
NCH = PER_W // CHUNK
CW = CHUNK * D
HW = CW // 2
FS = B * D
DHS = B * D // 2

_mesh = plsc.VectorSubcoreMesh(core_axis_name="c", subcore_axis_name="s")


@functools.partial(
    pl.kernel,
    out_type=jax.ShapeDtypeStruct((BF * D,), jnp.float32),
    mesh=_mesh,
    scratch_types=[
        pltpu.VMEM((PER_W,), jnp.int32),
        pltpu.VMEM((32,), jnp.int32),
        pltpu.VMEM((4 * CHUNK, D), jnp.float32),
        pltpu.VMEM((2 * CW,), jnp.float32),
        pltpu.SemaphoreType.DMA,
        pltpu.SemaphoreType.DMA,
    ],
    compiler_params=pltpu.CompilerParams(
        use_tc_tiling_on_sc=False, needs_layout_passes=False
    ),
)
def _embed(x_hbm, table_hbm, offs_hbm, out_hbm, xv, offs_v, big_v, stage_v,
           gsem, osem):
    wid = lax.axis_index("s") * NC + lax.axis_index("c")
    base = wid * PER_W

    pltpu.sync_copy(x_hbm.at[pl.ds(base, PER_W)], xv)
    pltpu.sync_copy(offs_hbm, offs_v)

    def add_body(i, carry):
        f = lax.shift_right_logical(base + i * L, 14)
        off = plsc.load_gather(offs_v, [lax.broadcast(f, (L,))])
        s = pl.ds(i * L, L)
        xv[s] = xv[s] + off
        return carry

    lax.fori_loop(0, PER_W // L, add_body, 0)

    iota = lax.iota(jnp.int32, L)
    cols = [lax.broadcast(jnp.int32(d), (L,)) for d in range(D)]

    def bigbuf(c):
        return big_v.at[pl.ds(lax.rem(c, 4) * CHUNK, CHUNK)]

    def fire(c):
        pltpu.async_copy(
            table_hbm.at[xv.at[pl.ds(c * CHUNK, CHUNK)]], bigbuf(c), gsem
        )

    def drain_gather(c):
        pltpu.make_async_copy(
            table_hbm.at[pl.ds(0, CHUNK)], bigbuf(c), gsem
        ).wait()

    def extract(c):
        src = bigbuf(c)
        sbuf = lax.rem(c, 2) * CW
        del c

        def block(b, carry):
            row = iota + b * L
            sb = sbuf + b * L
            for d in range(D):
                vals = plsc.load_gather(src, [row, cols[d]])
                stage_v[pl.ds(sb + d * CHUNK, L)] = vals
            return carry

        lax.fori_loop(0, CHUNK // L, block, 0)

    def out_half(c, dh):
        j0 = base + c * CHUNK
        f = lax.shift_right_logical(j0, 14)
        bt = lax.shift_right_logical(lax.rem(j0, B), 7)
        return out_hbm.at[pl.ds(f * FS + dh * DHS + bt * HW, HW)]

    def stage_half(c, dh):
        return stage_v.at[pl.ds(lax.rem(c, 2) * CW + dh * HW, HW)]

    fire(0)
    fire(1)
    fire(2)

    def chunk_body(c, carry):
        @pl.when(c + 3 < NCH)
        def _():
            fire(c + 3)

        @pl.when(c >= 2)
        def _():
            pltpu.make_async_copy(stage_half(c, 0), out_half(c, 0), osem).wait()
            pltpu.make_async_copy(stage_half(c, 1), out_half(c, 1), osem).wait()

        drain_gather(c)
        extract(c)
        pltpu.async_copy(stage_half(c, 0), out_half(c, 0), osem)
        pltpu.async_copy(stage_half(c, 1), out_half(c, 1), osem)
        return carry

    lax.fori_loop(0, NCH, chunk_body, 0)

    for c in (NCH - 2, NCH - 1):
        pltpu.make_async_copy(stage_half(c, 0), out_half(c, 0), osem).wait()
        pltpu.make_async_copy(stage_half(c, 1), out_half(c, 1), osem).wait()


def kernel(x_cat, table, offsets):
    xT = jnp.transpose(x_cat).reshape(BF)
    offs_pad = jnp.concatenate([offsets, jnp.zeros((32 - F,), jnp.int32)])
    out = _embed(xT, table, offs_pad)
    out5 = out.reshape(F, 2, B // 128, D // 2, 128)
    return out5.transpose(2, 4, 0, 1, 3).reshape(B, F, D)


# speedup vs baseline: 1.8116x; 1.0031x over previous
import functools

import jax
import jax.numpy as jnp
from jax import lax
from jax.experimental import pallas as pl
from jax.experimental.pallas import tpu as pltpu
from jax.experimental.pallas import tpu_sc as plsc

B = 16384
F = 26
D = 16
BF = B * F
NC, NS, L = 2, 16, 16
NW = NC * NS
PER_W = BF // NW
CHUNK = 128
NCH = PER_W // CHUNK
CW = CHUNK * D
HW = CW // 2
FS = B * D
DHS = B * D // 2

_mesh = plsc.VectorSubcoreMesh(core_axis_name="c", subcore_axis_name="s")


@functools.partial(
    pl.kernel,
    out_type=jax.ShapeDtypeStruct((BF * D,), jnp.float32),
    mesh=_mesh,
    scratch_types=[
        pltpu.VMEM((PER_W,), jnp.int32),
        pltpu.VMEM((32,), jnp.int32),
        pltpu.VMEM((4 * CHUNK, D), jnp.float32),
        pltpu.VMEM((2 * CW,), jnp.float32),
        pltpu.SemaphoreType.DMA,
        pltpu.SemaphoreType.DMA,
    ],
    compiler_params=pltpu.CompilerParams(
        use_tc_tiling_on_sc=False, needs_layout_passes=False
    ),
)
def _embed(x_hbm, table_hbm, offs_hbm, out_hbm, xv, offs_v, big_v, stage_v,
           gsem, osem):
    wid = lax.axis_index("s") * NC + lax.axis_index("c")
    base = wid * PER_W

    pltpu.sync_copy(x_hbm.at[pl.ds(base, PER_W)], xv)
    pltpu.sync_copy(offs_hbm, offs_v)

    def add_body(c, carry):
        f = lax.shift_right_logical(base + c * CHUNK, 14)
        off = plsc.load_gather(offs_v, [lax.broadcast(f, (L,))])
        for k in range(CHUNK // L):
            s = pl.ds(c * CHUNK + k * L, L)
            xv[s] = xv[s] + off
        return carry

    lax.fori_loop(0, NCH, add_body, 0)

    iota = lax.iota(jnp.int32, L)
    cols = [lax.broadcast(jnp.int32(d), (L,)) for d in range(D)]

    def bigbuf(c):
        return big_v.at[pl.ds(lax.rem(c, 4) * CHUNK, CHUNK)]

    def fire(c):
        pltpu.async_copy(
            table_hbm.at[xv.at[pl.ds(c * CHUNK, CHUNK)]], bigbuf(c), gsem
        )

    def drain_gather(c):
        pltpu.make_async_copy(
            table_hbm.at[pl.ds(0, CHUNK)], bigbuf(c), gsem
        ).wait()

    def extract(c):
        src = bigbuf(c)
        sbuf = lax.rem(c, 2) * CW
        del c

        for b in range(CHUNK // L):
            row = iota + b * L
            sb = sbuf + b * L
            for d in range(D):
                vals = plsc.load_gather(src, [row, cols[d]])
                stage_v[pl.ds(sb + d * CHUNK, L)] = vals

    def out_half(c, dh):
        j0 = base + c * CHUNK
        f = lax.shift_right_logical(j0, 14)
        bt = lax.shift_right_logical(lax.rem(j0, B), 7)
        return out_hbm.at[pl.ds(f * FS + dh * DHS + bt * HW, HW)]

    def stage_half(c, dh):
        return stage_v.at[pl.ds(lax.rem(c, 2) * CW + dh * HW, HW)]

    fire(0)
    fire(1)
    fire(2)

    def chunk_body(c, carry):
        @pl.when(c + 3 < NCH)
        def _():
            fire(c + 3)

        @pl.when(c >= 2)
        def _():
            pltpu.make_async_copy(stage_half(c, 0), out_half(c, 0), osem).wait()
            pltpu.make_async_copy(stage_half(c, 1), out_half(c, 1), osem).wait()

        drain_gather(c)
        extract(c)
        pltpu.async_copy(stage_half(c, 0), out_half(c, 0), osem)
        pltpu.async_copy(stage_half(c, 1), out_half(c, 1), osem)
        return carry

    lax.fori_loop(0, NCH, chunk_body, 0)

    for c in (NCH - 2, NCH - 1):
        pltpu.make_async_copy(stage_half(c, 0), out_half(c, 0), osem).wait()
        pltpu.make_async_copy(stage_half(c, 1), out_half(c, 1), osem).wait()


def kernel(x_cat, table, offsets):
    xT = jnp.transpose(x_cat).reshape(BF)
    offs_pad = jnp.concatenate([offsets, jnp.zeros((32 - F,), jnp.int32)])
    out = _embed(xT, table, offs_pad)
    out5 = out.reshape(F, 2, B // 128, D // 2, 128)
    return out5.transpose(2, 4, 0, 1, 3).reshape(B, F, D)
